# baseline (speedup 1.0000x reference)
import functools

import jax
import jax.numpy as jnp
from jax import lax
from jax.experimental import pallas as pl
from jax.experimental.pallas import tpu as pltpu
from jax.experimental.pallas import tpu_sc as plsc

_VOCAB = 100000
_NE = 768
_T = 2048
_B = 4
_LANES = 16
_NC = 2
_NS = 16
_NW = _NC * _NS
_ROWS = _B * _T
_ROWS_PER_W = _ROWS // _NW
_WPB = _T // _ROWS_PER_W
_CHUNK = 16
_NCHUNK = _ROWS_PER_W // _CHUNK


def _emb_body(idx_hbm, tok_hbm, pos_hbm, out_hbm,
              idx_v, mask_v, rows0, rows1, pos0, pos1, pos2,
              gs0, gs1, ps0, ps1, ps2, os0, os1, os2, isem):
    rows = (rows0, rows1)
    pos = (pos0, pos1, pos2)
    gsem = (gs0, gs1)
    psem = (ps0, ps1, ps2)
    osem = (os0, os1, os2)

    c = lax.axis_index("c")
    s = lax.axis_index("s")
    wid = s * _NC + c
    base = wid * _ROWS_PER_W
    bat = wid // _WPB
    t0 = (wid % _WPB) * _ROWS_PER_W

    def pissue(ck):
        return pltpu.async_copy(
            pos_hbm.at[pl.ds(t0 + ck * _CHUNK, _CHUNK)], pos[ck % 3],
            psem[ck % 3])

    pcp = {0: pissue(0)}
    icopy = pltpu.async_copy(idx_hbm.at[bat, pl.ds(t0, _ROWS_PER_W)],
                             idx_v, isem)
    icopy.wait()

    for j in range(_ROWS_PER_W // _LANES):
        sl = pl.ds(j * _LANES, _LANES)
        v = idx_v[sl]
        bad = jnp.logical_or(v < 1, v > _VOCAB)
        idx_v[sl] = jnp.where(bad, 0, v)
        mask_v[sl] = jnp.where(bad, 0.0, 1.0).astype(jnp.float32)

    def gissue(ck):
        return pltpu.async_copy(
            tok_hbm.at[idx_v.at[pl.ds(ck * _CHUNK, _CHUNK)]], rows[ck % 2],
            gsem[ck % 2])

    gcp = {0: gissue(0)}
    outcp = {}
    for ck in range(_NCHUNK):
        b = ck % 2
        p3 = ck % 3
        if ck + 1 < _NCHUNK:
            gcp[ck + 1] = gissue(ck + 1)
            if ck - 2 >= 0:
                outcp[ck - 2].wait()
            pcp[ck + 1] = pissue(ck + 1)
        gcp[ck].wait()
        pcp[ck].wait()

        def row_body(r, carry, ck=ck, b=b, p3=p3):
            m = mask_v[pl.ds(ck * _CHUNK + r, _LANES)][0]

            @pl.when(m != 0.0)
            def _():
                for j in range(_NE // _LANES):
                    sl = pl.ds(j * _LANES, _LANES)
                    plsc.addupdate(pos[p3].at[r, sl], rows[b][r, sl])

            return carry

        lax.fori_loop(0, _CHUNK, row_body, 0)
        outcp[ck] = pltpu.async_copy(
            pos[p3], out_hbm.at[pl.ds(base + ck * _CHUNK, _CHUNK)], osem[p3])
    outcp[_NCHUNK - 3].wait()
    outcp[_NCHUNK - 2].wait()
    outcp[_NCHUNK - 1].wait()


@jax.jit
def _emb_call(idx2d, tok_emb, pos2d):
    mesh = plsc.VectorSubcoreMesh(core_axis_name="c", subcore_axis_name="s")
    kfn = pl.kernel(
        _emb_body,
        mesh=mesh,
        out_type=jax.ShapeDtypeStruct((_ROWS, _NE), jnp.float32),
        scratch_types=[
            pltpu.VMEM((_ROWS_PER_W,), jnp.int32),
            pltpu.VMEM((_ROWS_PER_W + _LANES,), jnp.float32),
            pltpu.VMEM((_CHUNK, _NE), jnp.float32),
            pltpu.VMEM((_CHUNK, _NE), jnp.float32),
            pltpu.VMEM((_CHUNK, _NE), jnp.float32),
            pltpu.VMEM((_CHUNK, _NE), jnp.float32),
            pltpu.VMEM((_CHUNK, _NE), jnp.float32),
            pltpu.SemaphoreType.DMA,
            pltpu.SemaphoreType.DMA,
            pltpu.SemaphoreType.DMA,
            pltpu.SemaphoreType.DMA,
            pltpu.SemaphoreType.DMA,
            pltpu.SemaphoreType.DMA,
            pltpu.SemaphoreType.DMA,
            pltpu.SemaphoreType.DMA,
            pltpu.SemaphoreType.DMA,
        ],
    )
    return kfn(idx2d, tok_emb, pos2d)


def kernel(idx, tok_emb, pos_emb):
    b, t = idx.shape
    out = _emb_call(idx, tok_emb, pos_emb.reshape(-1, _NE)[:t])
    return out.reshape(b, t, _NE)


# speedup vs baseline: 1.0163x; 1.0163x over previous
import functools

import jax
import jax.numpy as jnp
from jax import lax
from jax.experimental import pallas as pl
from jax.experimental.pallas import tpu as pltpu
from jax.experimental.pallas import tpu_sc as plsc

_VOCAB = 100000
_NE = 768
_T = 2048
_B = 4
_LANES = 16
_NC = 2
_NS = 16
_NW = _NC * _NS
_ROWS = _B * _T
_ROWS_PER_W = _ROWS // _NW
_WPB = _T // _ROWS_PER_W
_CHUNK = 32
_NCHUNK = _ROWS_PER_W // _CHUNK


def _emb_body(idx_hbm, tok_hbm, pos_hbm, out_hbm,
              idx_v, mask_v, rows0, rows1, pos0, pos1, pos2,
              gs0, gs1, ps0, ps1, ps2, os0, os1, os2, isem):
    rows = (rows0, rows1)
    pos = (pos0, pos1, pos2)
    gsem = (gs0, gs1)
    psem = (ps0, ps1, ps2)
    osem = (os0, os1, os2)

    c = lax.axis_index("c")
    s = lax.axis_index("s")
    wid = s * _NC + c
    base = wid * _ROWS_PER_W
    bat = wid // _WPB
    t0 = (wid % _WPB) * _ROWS_PER_W

    def pissue(ck):
        return pltpu.async_copy(
            pos_hbm.at[pl.ds(t0 + ck * _CHUNK, _CHUNK)], pos[ck % 3],
            psem[ck % 3])

    pcp = {0: pissue(0)}
    icopy = pltpu.async_copy(idx_hbm.at[bat, pl.ds(t0, _ROWS_PER_W)],
                             idx_v, isem)
    icopy.wait()

    for j in range(_ROWS_PER_W // _LANES):
        sl = pl.ds(j * _LANES, _LANES)
        v = idx_v[sl]
        bad = jnp.logical_or(v < 1, v > _VOCAB)
        idx_v[sl] = jnp.where(bad, 0, v)
        mask_v[sl] = jnp.where(bad, 0.0, 1.0).astype(jnp.float32)

    def gissue(ck):
        return pltpu.async_copy(
            tok_hbm.at[idx_v.at[pl.ds(ck * _CHUNK, _CHUNK)]], rows[ck % 2],
            gsem[ck % 2])

    gcp = {0: gissue(0)}
    outcp = {}
    for ck in range(_NCHUNK):
        b = ck % 2
        p3 = ck % 3
        if ck + 1 < _NCHUNK:
            gcp[ck + 1] = gissue(ck + 1)
            if ck - 2 >= 0:
                outcp[ck - 2].wait()
            pcp[ck + 1] = pissue(ck + 1)
        gcp[ck].wait()
        pcp[ck].wait()

        def row_body(r, carry, ck=ck, b=b, p3=p3):
            m = mask_v[pl.ds(ck * _CHUNK + r, _LANES)][0]

            @pl.when(m != 0.0)
            def _():
                for j in range(_NE // _LANES):
                    sl = pl.ds(j * _LANES, _LANES)
                    plsc.addupdate(pos[p3].at[r, sl], rows[b][r, sl])

            return carry

        lax.fori_loop(0, _CHUNK, row_body, 0)
        outcp[ck] = pltpu.async_copy(
            pos[p3], out_hbm.at[pl.ds(base + ck * _CHUNK, _CHUNK)], osem[p3])
    outcp[_NCHUNK - 3].wait()
    outcp[_NCHUNK - 2].wait()
    outcp[_NCHUNK - 1].wait()


@jax.jit
def _emb_call(idx2d, tok_emb, pos2d):
    mesh = plsc.VectorSubcoreMesh(core_axis_name="c", subcore_axis_name="s")
    kfn = pl.kernel(
        _emb_body,
        mesh=mesh,
        out_type=jax.ShapeDtypeStruct((_ROWS, _NE), jnp.float32),
        scratch_types=[
            pltpu.VMEM((_ROWS_PER_W,), jnp.int32),
            pltpu.VMEM((_ROWS_PER_W + _LANES,), jnp.float32),
            pltpu.VMEM((_CHUNK, _NE), jnp.float32),
            pltpu.VMEM((_CHUNK, _NE), jnp.float32),
            pltpu.VMEM((_CHUNK, _NE), jnp.float32),
            pltpu.VMEM((_CHUNK, _NE), jnp.float32),
            pltpu.VMEM((_CHUNK, _NE), jnp.float32),
            pltpu.SemaphoreType.DMA,
            pltpu.SemaphoreType.DMA,
            pltpu.SemaphoreType.DMA,
            pltpu.SemaphoreType.DMA,
            pltpu.SemaphoreType.DMA,
            pltpu.SemaphoreType.DMA,
            pltpu.SemaphoreType.DMA,
            pltpu.SemaphoreType.DMA,
            pltpu.SemaphoreType.DMA,
        ],
    )
    return kfn(idx2d, tok_emb, pos2d)


def kernel(idx, tok_emb, pos_emb):
    b, t = idx.shape
    out = _emb_call(idx, tok_emb, pos_emb.reshape(-1, _NE)[:t])
    return out.reshape(b, t, _NE)
